# Initial kernel scaffold; baseline (speedup 1.0000x reference)
import jax
import jax.numpy as jnp
from jax.experimental import pallas as pl


def kernel(X, vertex_ids, hyperedge_ids, W1, b1, W2, b2):
    raise NotImplementedError("write your pallas kernel here")


# speedup vs baseline: 3.6739x; 3.6739x over previous
import functools

import jax
import jax.numpy as jnp
from jax import lax
from jax.experimental import pallas as pl
from jax.experimental.pallas import tpu as pltpu
from jax.experimental.pallas import tpu_sc as plsc

N = 10000
D = 128
DP = 144
R = 10240
NNZ = 320000
NW = 32
K = 128
CH = 79
NNZ_PAD = NW * CH * K
TRASH = N
STRIPE = R // 16


def _seg_body(table, src_ids, dst_ids, out, acc,
              idx_s0, idx_s1, idx_d0, idx_d1, rows0, rows1,
              gsem0, gsem1, ssem0, ssem1):
    c = lax.axis_index("c")
    s = lax.axis_index("s")
    w = c * 16 + s

    idx_s = (idx_s0, idx_s1)
    idx_d = (idx_d0, idx_d1)
    rows = (rows0, rows1)
    gsem = (gsem0, gsem1)
    ssem = (ssem0, ssem1)

    def _zrow(i, _):
        for cc in range(DP // 16):
            rows0[i, pl.ds(cc * 16, 16)] = jnp.zeros((16,), jnp.float32)
        return _
    lax.fori_loop(0, K, _zrow, None)
    for t in range(STRIPE // K):
        pltpu.sync_copy(rows0, acc.at[pl.ds(s * STRIPE + t * K, K), :])
    plsc.subcore_barrier()

    def _chunk(j, b):
        @pl.when(j >= 2)
        def _():
            pltpu.make_async_copy(rows[b], acc.at[idx_d[b]], ssem[b]).wait()
        base = (w * CH + j) * K
        pltpu.sync_copy(src_ids.at[pl.ds(base, K)], idx_s[b])
        pltpu.sync_copy(dst_ids.at[pl.ds(base, K)], idx_d[b])
        pltpu.async_copy(table.at[idx_s[b]], rows[b], gsem[b]).wait()
        pltpu.async_copy(rows[b], acc.at[idx_d[b]], ssem[b], add=True)

    def _pair(i, _):
        jj = 2 * i
        _chunk(jj, 0)
        _chunk(jj + 1, 1)
        return _
    lax.fori_loop(0, (CH - 1) // 2, _pair, None)
    _chunk(CH - 1, 0)
    pltpu.make_async_copy(rows[0], acc.at[idx_d[0]], ssem[0]).wait()
    pltpu.make_async_copy(rows[1], acc.at[idx_d[1]], ssem[1]).wait()

    plsc.subcore_barrier()
    for t in range(STRIPE // K):
        base = s * STRIPE + t * K
        pltpu.sync_copy(acc.at[pl.ds(base, K), :], rows0)
        pltpu.sync_copy(rows0, out.at[c, pl.ds(base, K), :])


_seg = pl.kernel(
    _seg_body,
    out_type=jax.ShapeDtypeStruct((2, R, DP), jnp.float32),
    mesh=plsc.VectorSubcoreMesh(core_axis_name="c", subcore_axis_name="s"),
    scratch_types=[
        pltpu.VMEM_SHARED((R, DP), jnp.float32),
        pltpu.VMEM((K,), jnp.int32),
        pltpu.VMEM((K,), jnp.int32),
        pltpu.VMEM((K,), jnp.int32),
        pltpu.VMEM((K,), jnp.int32),
        pltpu.VMEM((K, DP), jnp.float32),
        pltpu.VMEM((K, DP), jnp.float32),
        pltpu.SemaphoreType.DMA,
        pltpu.SemaphoreType.DMA,
        pltpu.SemaphoreType.DMA,
        pltpu.SemaphoreType.DMA,
    ],
    compiler_params=pltpu.CompilerParams(use_tc_tiling_on_sc=False),
)


_BR = 512


def _mm_pad_body(x_ref, w_ref, b_ref, o_ref):
    h = jnp.dot(x_ref[...], w_ref[...],
                preferred_element_type=jnp.float32) + b_ref[...]
    o_ref[...] = jnp.concatenate(
        [h, jnp.ones((h.shape[0], 1), jnp.float32),
         jnp.zeros((h.shape[0], DP - D - 1), jnp.float32)], axis=1)


def _mm_pad(xp, w, b):
    return pl.pallas_call(
        _mm_pad_body,
        grid=(R // _BR,),
        in_specs=[
            pl.BlockSpec((_BR, D), lambda i: (i, 0)),
            pl.BlockSpec((D, D), lambda i: (0, 0)),
            pl.BlockSpec((1, D), lambda i: (0, 0)),
        ],
        out_specs=pl.BlockSpec((_BR, DP), lambda i: (i, 0)),
        out_shape=jax.ShapeDtypeStruct((R, DP), jnp.float32),
    )(xp, w, b)


def _comb_body(p_ref, o_ref):
    t = p_ref[0] + p_ref[1]
    d = jnp.maximum(t[:, D:D + 1], 1.0)
    o_ref[...] = t / d


def _comb(p):
    return pl.pallas_call(
        _comb_body,
        grid=(R // _BR,),
        in_specs=[pl.BlockSpec((2, _BR, DP), lambda i: (0, i, 0))],
        out_specs=pl.BlockSpec((_BR, DP), lambda i: (i, 0)),
        out_shape=jax.ShapeDtypeStruct((R, DP), jnp.float32),
    )(p)


def _relu_mm_pad_body(p_ref, w_ref, b_ref, o_ref):
    t = p_ref[0] + p_ref[1]
    d = jnp.maximum(t[:, D:D + 1], 1.0)
    xv = jnp.maximum(t[:, :D] / d, 0.0)
    h = jnp.dot(xv, w_ref[...], preferred_element_type=jnp.float32) + b_ref[...]
    o_ref[...] = jnp.concatenate(
        [h, jnp.ones((h.shape[0], 1), jnp.float32),
         jnp.zeros((h.shape[0], DP - D - 1), jnp.float32)], axis=1)


def _relu_mm_pad(p, w, b):
    return pl.pallas_call(
        _relu_mm_pad_body,
        grid=(R // _BR,),
        in_specs=[
            pl.BlockSpec((2, _BR, DP), lambda i: (0, i, 0)),
            pl.BlockSpec((D, D), lambda i: (0, 0)),
            pl.BlockSpec((1, D), lambda i: (0, 0)),
        ],
        out_specs=pl.BlockSpec((_BR, DP), lambda i: (i, 0)),
        out_shape=jax.ShapeDtypeStruct((R, DP), jnp.float32),
    )(p, w, b)


_FBR = 400


def _final_body(p_ref, o_ref):
    t = p_ref[0] + p_ref[1]
    d = jnp.maximum(t[:, D:D + 1], 1.0)
    o_ref[...] = jnp.maximum(t[:, :D] / d, 0.0)


def _final(p):
    return pl.pallas_call(
        _final_body,
        grid=(N // _FBR,),
        in_specs=[pl.BlockSpec((2, _FBR, DP), lambda i: (0, i, 0))],
        out_specs=pl.BlockSpec((_FBR, D), lambda i: (i, 0)),
        out_shape=jax.ShapeDtypeStruct((N, D), jnp.float32),
    )(p)


def kernel(X, vertex_ids, hyperedge_ids, W1, b1, W2, b2):
    xp = jnp.zeros((R, D), jnp.float32).at[:N].set(X)
    pad = jnp.full((NNZ_PAD - NNZ,), TRASH, jnp.int32)
    vv = jnp.concatenate([vertex_ids, pad])
    ee = jnp.concatenate([hyperedge_ids, pad])
    b1r = b1.reshape(1, D)
    b2r = b2.reshape(1, D)

    h = _mm_pad(xp, W1, b1r)
    p = _seg(h, vv, ee)
    xe = _comb(p)
    p = _seg(xe, ee, vv)
    h = _relu_mm_pad(p, W2, b2r)
    p = _seg(h, vv, ee)
    xe = _comb(p)
    p = _seg(xe, ee, vv)
    return _final(p)
